# baseline (speedup 1.0000x reference)
import jax
import jax.numpy as jnp
from jax import lax
from jax.experimental import pallas as pl
from jax.experimental.pallas import tpu as pltpu

ROWS = 100000
COLS = 512
N_TOTAL = float(ROWS * COLS)

SMALL = 400
BIG = 2400
N_HEAD = 5
N_BIG = 40
N_TAIL = 5
N_CHUNKS = N_HEAD + N_BIG + N_TAIL


def _row0(c):
    if isinstance(c, int):
        if c < N_HEAD:
            return c * SMALL
        if c < N_HEAD + N_BIG:
            return 2000 + (c - N_HEAD) * BIG
        return 98000 + (c - N_HEAD - N_BIG) * SMALL
    return 2000 + (c - N_HEAD) * BIG


def _nrows(c):
    return BIG if N_HEAD <= c < N_HEAD + N_BIG else SMALL


def _body(o_hbm, t_hbm, m_hbm, loss_ref, bo, bt, bm, sems, acc_ref):
    def copies(c, nrows, slot):
        r0 = _row0(c)
        return [
            pltpu.make_async_copy(
                o_hbm.at[pl.ds(r0, nrows)], bo.at[slot, pl.ds(0, nrows)],
                sems.at[slot, 0],
            ),
            pltpu.make_async_copy(
                t_hbm.at[pl.ds(r0, nrows)], bt.at[slot, pl.ds(0, nrows)],
                sems.at[slot, 1],
            ),
            pltpu.make_async_copy(
                m_hbm.at[pl.ds(r0, nrows)], bm.at[slot, pl.ds(0, nrows)],
                sems.at[slot, 2],
            ),
        ]

    def issue(c, nrows, slot):
        for cp in copies(c, nrows, slot):
            cp.start()

    def wait(c, nrows, slot):
        for cp in copies(c, nrows, slot):
            cp.wait()

    def accumulate(nrows, slot):
        o = bo[slot, pl.ds(0, nrows), :]
        t = bt[slot, pl.ds(0, nrows), :]
        m = bm[slot, pl.ds(0, nrows), :].astype(jnp.float32)
        d = o - t
        d2 = d * d
        acc_ref[0] += jnp.sum(d2 * m)
        acc_ref[1] += jnp.sum(d2)
        acc_ref[2] += jnp.sum(m)

    acc_ref[0] = 0.0
    acc_ref[1] = 0.0
    acc_ref[2] = 0.0

    issue(0, SMALL, 0)
    issue(1, SMALL, 1)

    for c in range(N_HEAD):
        issue(c + 2, _nrows(c + 2), (c + 2) % 3)
        wait(c, SMALL, c % 3)
        accumulate(SMALL, c % 3)

    def loop_body(j, _):
        for k, slot in ((0, 2), (1, 0), (2, 1)):
            c = 5 + 3 * j + k
            issue(c + 2, BIG, (slot + 2) % 3)
            wait(c, BIG, slot)
            accumulate(BIG, slot)
        return 0

    lax.fori_loop(0, 12, loop_body, 0)

    for c in (41, 42, 43, 44):
        issue(c + 2, _nrows(c + 2), (c + 2) % 3)
        wait(c, BIG, c % 3)
        accumulate(BIG, c % 3)

    for c in range(45, N_CHUNKS):
        if c + 2 < N_CHUNKS:
            issue(c + 2, SMALL, (c + 2) % 3)
        wait(c, SMALL, c % 3)
        accumulate(SMALL, c % 3)

    s_m = acc_ref[0]
    s_tot = acc_ref[1]
    cnt = acc_ref[2]
    loss_ref[0] = s_m / jnp.maximum(cnt, 1.0) + (s_tot - s_m) / jnp.maximum(
        N_TOTAL - cnt, 1.0
    )


def kernel(output, target, mask):
    loss = pl.pallas_call(
        _body,
        in_specs=[
            pl.BlockSpec(memory_space=pl.ANY),
            pl.BlockSpec(memory_space=pl.ANY),
            pl.BlockSpec(memory_space=pl.ANY),
        ],
        out_specs=pl.BlockSpec(memory_space=pltpu.SMEM),
        out_shape=jax.ShapeDtypeStruct((1,), jnp.float32),
        scratch_shapes=[
            pltpu.VMEM((3, BIG, COLS), jnp.float32),
            pltpu.VMEM((3, BIG, COLS), jnp.float32),
            pltpu.VMEM((3, BIG, COLS), jnp.int32),
            pltpu.SemaphoreType.DMA((3, 3)),
            pltpu.SMEM((3,), jnp.float32),
        ],
    )(output, target, mask)
    return loss[0]


# speedup vs baseline: 1.0065x; 1.0065x over previous
import jax
import jax.numpy as jnp
from jax.experimental import pallas as pl
from jax.experimental.pallas import tpu as pltpu

ROWS = 100000
COLS = 512
BLOCK_ROWS = 2000
NUM_BLOCKS = ROWS // BLOCK_ROWS
N_TOTAL = float(ROWS * COLS)


def _body(o_ref, t_ref, m_ref, loss_ref, acc_ref):
    i = pl.program_id(0)

    d = o_ref[...] - t_ref[...]
    d2 = d * d
    m = m_ref[...].astype(jnp.float32)

    psum_m = jnp.sum(d2 * m)
    psum_tot = jnp.sum(d2)
    pcnt = jnp.sum(m)

    @pl.when(i == 0)
    def _init():
        acc_ref[0] = 0.0
        acc_ref[1] = 0.0
        acc_ref[2] = 0.0

    acc_ref[0] += psum_m
    acc_ref[1] += psum_tot
    acc_ref[2] += pcnt

    @pl.when(i == NUM_BLOCKS - 1)
    def _final():
        s_m = acc_ref[0]
        s_tot = acc_ref[1]
        c = acc_ref[2]
        loss = s_m / jnp.maximum(c, 1.0) + (s_tot - s_m) / jnp.maximum(
            N_TOTAL - c, 1.0
        )
        loss_ref[0, 0] = loss


def kernel(output, target, mask):
    loss = pl.pallas_call(
        _body,
        grid=(NUM_BLOCKS,),
        in_specs=[
            pl.BlockSpec((BLOCK_ROWS, COLS), lambda i: (i, 0)),
            pl.BlockSpec((BLOCK_ROWS, COLS), lambda i: (i, 0)),
            pl.BlockSpec((BLOCK_ROWS, COLS), lambda i: (i, 0)),
        ],
        out_specs=pl.BlockSpec(
            (1, 1), lambda i: (0, 0), memory_space=pltpu.SMEM
        ),
        out_shape=jax.ShapeDtypeStruct((1, 1), jnp.float32),
        scratch_shapes=[pltpu.SMEM((3,), jnp.float32)],
    )(output, target, mask)
    return loss[0, 0]
